# baseline (speedup 1.0000x reference)
import functools

import jax
import jax.numpy as jnp
from jax import lax
from jax.experimental import pallas as pl
from jax.experimental.pallas import tpu as pltpu
from jax.experimental.pallas import tpu_sc as plsc

L = 16
NC = 2
NS = 16
NW = NC * NS

_CP = pltpu.CompilerParams(needs_layout_passes=False, skip_device_barrier=True)


def _make_fused(N, B, T, C, K):
    mesh = plsc.VectorSubcoreMesh(core_axis_name="c", subcore_axis_name="s")
    V = C // L
    NBUF = 4

    @functools.partial(
        pl.kernel,
        out_type=(
            jax.ShapeDtypeStruct((N,), jnp.float32),
            jax.ShapeDtypeStruct((B,), jnp.float32),
            jax.ShapeDtypeStruct((NC * B,), jnp.float32),
            jax.ShapeDtypeStruct((NC * B,), jnp.float32),
        ),
        mesh=mesh,
        compiler_params=_CP,
        scratch_types=[
            *[pltpu.VMEM((C,), jnp.float32) for _ in range(NBUF)],
            *[pltpu.VMEM((C,), jnp.int32) for _ in range(NBUF)],
            pltpu.VMEM((B,), jnp.float32),
            pltpu.VMEM((B,), jnp.float32),
            pltpu.VMEM((B,), jnp.int32),
            pltpu.VMEM_SHARED((B,), jnp.float32),
            pltpu.VMEM_SHARED((B,), jnp.float32),
            pltpu.VMEM((NC * B,), jnp.float32),
            pltpu.VMEM((NC * B,), jnp.float32),
            pltpu.VMEM((B,), jnp.float32),
            pltpu.VMEM((B,), jnp.float32),
            pltpu.VMEM((B,), jnp.float32),
            *[pltpu.SemaphoreType.DMA for _ in range(NBUF)],
            pltpu.SemaphoreType.DMA,
            pltpu.SemaphoreType.REGULAR,
        ],
    )
    def fused(qa_hbm, seg_hbm, q_hbm, out_hbm, rawq_hbm, psum_hbm, pcnt_hbm,
              *refs):
        qa_bufs = refs[0:NBUF]
        seg_bufs = refs[NBUF:2 * NBUF]
        (acc_s, acc_c, idx, sh_s, sh_c,
         ps, pc, qv, corr, raw) = refs[2 * NBUF:2 * NBUF + 10]
        sems = refs[2 * NBUF + 10:2 * NBUF + 10 + NBUF]
        semp = refs[2 * NBUF + 10 + NBUF]
        xsem = refs[2 * NBUF + 10 + NBUF + 1]

        cid = lax.axis_index("c")
        sid = lax.axis_index("s")
        wid = cid * NS + sid
        base = wid * T

        zz = jnp.zeros((L,), jnp.float32)
        lane = lax.iota(jnp.int32, L)

        @plsc.parallel_loop(0, B // L, unroll=4)
        def _zero(j):
            acc_s[pl.ds(j * L, L)] = zz
            acc_c[pl.ds(j * L, L)] = zz
            idx[pl.ds(j * L, L)] = lane + j * L

        @pl.when(sid == 0)
        def _():
            pltpu.sync_copy(acc_s, sh_s)
            pltpu.sync_copy(acc_c, sh_c)

        def start(k):
            b = k % NBUF
            return (
                pltpu.async_copy(qa_hbm.at[pl.ds(base + k * C, C)],
                                 qa_bufs[b], sems[b]),
                pltpu.async_copy(seg_hbm.at[pl.ds(base + k * C, C)],
                                 seg_bufs[b], sems[b]),
            )

        descs = [None] * K
        for k in range(min(NBUF, K)):
            descs[k] = start(k)
        descs[0][0].wait()
        descs[0][1].wait()

        cur0 = seg_bufs[0][pl.ds(0, L)]
        carry = (jnp.zeros((L,), jnp.float32),
                 jnp.zeros((L,), jnp.float32),
                 cur0)

        for k in range(K):
            b = k % NBUF
            if k > 0:
                descs[k][0].wait()
                descs[k][1].wait()
            qa_r = qa_bufs[b]
            seg_r = seg_bufs[b]

            def step(i, c, qa_r=qa_r, seg_r=seg_r):
                run_s, run_c, cur = c
                sl = pl.ds(i * L, L)
                qa = qa_r[sl]
                seg = seg_r[sl]
                changed = seg != cur
                plsc.addupdate_scatter(acc_s, [cur], run_s, mask=changed)
                plsc.addupdate_scatter(acc_c, [cur], run_c, mask=changed)
                run_s = jnp.where(changed, qa, run_s + qa)
                run_c = jnp.where(changed, jnp.full((L,), 1.0, jnp.float32),
                                  run_c + 1.0)
                return run_s, run_c, seg

            carry = plsc.parallel_loop(0, V, unroll=8, carry=carry)(step)
            if k + NBUF < K:
                descs[k + NBUF] = start(k + NBUF)

        run_s, run_c, cur = carry
        plsc.addupdate_scatter(acc_s, [cur], run_s)
        plsc.addupdate_scatter(acc_c, [cur], run_c)

        def start2(k):
            b = k % 2
            return (
                pltpu.async_copy(qa_hbm.at[pl.ds(base + k * C, C)],
                                 qa_bufs[b], sems[b]),
                pltpu.async_copy(seg_hbm.at[pl.ds(base + k * C, C)],
                                 seg_bufs[b], sems[b]),
            )

        in_descs = [None] * K
        in_descs[0] = start2(0)
        if K > 1:
            in_descs[1] = start2(1)
        d3 = pltpu.async_copy(q_hbm, qv, semp)

        plsc.subcore_barrier()
        pltpu.sync_copy(acc_s, sh_s.at[idx], add=True)
        pltpu.sync_copy(acc_c, sh_c.at[idx], add=True)
        plsc.subcore_barrier()

        @pl.when(sid == 0)
        def _():
            pltpu.sync_copy(sh_s, psum_hbm.at[pl.ds(cid * B, B)])
            pltpu.sync_copy(sh_c, pcnt_hbm.at[pl.ds(cid * B, B)])

        plsc.subcore_barrier()
        pltpu.semaphore_signal(xsem, 1, core_index=1 - cid)
        pl.semaphore_wait(xsem, 1)

        d1 = pltpu.async_copy(psum_hbm, ps, semp)
        d2 = pltpu.async_copy(pcnt_hbm, pc, semp)
        d1.wait()
        d2.wait()
        d3.wait()

        @plsc.parallel_loop(0, B // L, unroll=4)
        def _comb(j):
            s = jnp.zeros((L,), jnp.float32)
            n = jnp.zeros((L,), jnp.float32)
            for t in range(NC):
                s = s + ps[pl.ds(j * L + t * B, L)]
                n = n + pc[pl.ds(j * L + t * B, L)]
            sl = pl.ds(j * L, L)
            corr[sl] = (qv[sl] - s) / n
            raw[sl] = s

        out_bufs = (qa_bufs[2], qa_bufs[3])
        osems = (sems[2], sems[3])
        out_descs = [None] * K
        for k in range(K):
            b = k % 2
            in_descs[k][0].wait()
            in_descs[k][1].wait()
            if k >= 2:
                out_descs[k - 2].wait()
            qa_r = qa_bufs[b]
            seg_r = seg_bufs[b]
            ob = out_bufs[b]

            @plsc.parallel_loop(0, V, unroll=8)
            def _apply(i, qa_r=qa_r, seg_r=seg_r, ob=ob):
                sl = pl.ds(i * L, L)
                seg = seg_r[sl]
                qa = qa_r[sl]
                c = plsc.load_gather(corr, [seg])
                ob[sl] = qa + c

            out_descs[k] = pltpu.async_copy(
                ob, out_hbm.at[pl.ds(base + k * C, C)], osems[b])
            if k + 2 < K:
                in_descs[k + 2] = start2(k + 2)

        if K >= 2:
            out_descs[K - 2].wait()
        out_descs[K - 1].wait()

        @pl.when(wid == 0)
        def _():
            pltpu.sync_copy(raw, rawq_hbm)

    return fused


def kernel(Za, Qa, Q, batch_seg):
    del Za
    N = Qa.shape[0]
    B = Q.shape[0]
    assert N % NW == 0
    T = N // NW

    C = 10000
    assert T % C == 0 and C % L == 0

    qa = Qa.astype(jnp.float32)
    seg = batch_seg.astype(jnp.int32)
    q = Q.astype(jnp.float32)

    out, raw_q, _, _ = _make_fused(N, B, T, C, T // C)(qa, seg, q)
    return (out, raw_q)


# speedup vs baseline: 1.0101x; 1.0101x over previous
import functools

import jax
import jax.numpy as jnp
from jax import lax
from jax.experimental import pallas as pl
from jax.experimental.pallas import tpu as pltpu
from jax.experimental.pallas import tpu_sc as plsc

L = 16
NC = 2
NS = 16
NW = NC * NS

_CP = pltpu.CompilerParams(needs_layout_passes=False)


def _make_fused(N, B, T, C, K):
    mesh = plsc.VectorSubcoreMesh(core_axis_name="c", subcore_axis_name="s")
    V = C // L
    NBUF = 4

    @functools.partial(
        pl.kernel,
        out_type=(
            jax.ShapeDtypeStruct((N,), jnp.float32),
            jax.ShapeDtypeStruct((B,), jnp.float32),
            jax.ShapeDtypeStruct((NC * B,), jnp.float32),
            jax.ShapeDtypeStruct((NC * B,), jnp.float32),
        ),
        mesh=mesh,
        compiler_params=_CP,
        scratch_types=[
            *[pltpu.VMEM((C,), jnp.float32) for _ in range(NBUF)],
            *[pltpu.VMEM((C,), jnp.int32) for _ in range(NBUF)],
            pltpu.VMEM((B,), jnp.float32),
            pltpu.VMEM((B,), jnp.float32),
            pltpu.VMEM((B,), jnp.int32),
            pltpu.VMEM_SHARED((B,), jnp.float32),
            pltpu.VMEM_SHARED((B,), jnp.float32),
            pltpu.VMEM((NC * B,), jnp.float32),
            pltpu.VMEM((NC * B,), jnp.float32),
            pltpu.VMEM((B,), jnp.float32),
            pltpu.VMEM((B,), jnp.float32),
            pltpu.VMEM((B,), jnp.float32),
            *[pltpu.SemaphoreType.DMA for _ in range(NBUF)],
            pltpu.SemaphoreType.DMA,
            pltpu.SemaphoreType.REGULAR,
        ],
    )
    def fused(qa_hbm, seg_hbm, q_hbm, out_hbm, rawq_hbm, psum_hbm, pcnt_hbm,
              *refs):
        qa_bufs = refs[0:NBUF]
        seg_bufs = refs[NBUF:2 * NBUF]
        (acc_s, acc_c, idx, sh_s, sh_c,
         ps, pc, qv, corr, raw) = refs[2 * NBUF:2 * NBUF + 10]
        sems = refs[2 * NBUF + 10:2 * NBUF + 10 + NBUF]
        semp = refs[2 * NBUF + 10 + NBUF]
        xsem = refs[2 * NBUF + 10 + NBUF + 1]

        cid = lax.axis_index("c")
        sid = lax.axis_index("s")
        wid = cid * NS + sid
        base = wid * T

        zz = jnp.zeros((L,), jnp.float32)
        lane = lax.iota(jnp.int32, L)

        @plsc.parallel_loop(0, B // L, unroll=4)
        def _zero(j):
            acc_s[pl.ds(j * L, L)] = zz
            acc_c[pl.ds(j * L, L)] = zz
            idx[pl.ds(j * L, L)] = lane + j * L

        @pl.when(sid == 0)
        def _():
            pltpu.sync_copy(acc_s, sh_s)
            pltpu.sync_copy(acc_c, sh_c)

        def start(k):
            b = k % NBUF
            return (
                pltpu.async_copy(qa_hbm.at[pl.ds(base + k * C, C)],
                                 qa_bufs[b], sems[b]),
                pltpu.async_copy(seg_hbm.at[pl.ds(base + k * C, C)],
                                 seg_bufs[b], sems[b]),
            )

        descs = [None] * K
        for k in range(min(NBUF, K)):
            descs[k] = start(k)
        descs[0][0].wait()
        descs[0][1].wait()

        cur0 = seg_bufs[0][pl.ds(0, L)]
        carry = (jnp.zeros((L,), jnp.float32),
                 jnp.zeros((L,), jnp.float32),
                 cur0)

        for k in range(K):
            b = k % NBUF
            if k > 0:
                descs[k][0].wait()
                descs[k][1].wait()
            qa_r = qa_bufs[b]
            seg_r = seg_bufs[b]

            def step(i, c, qa_r=qa_r, seg_r=seg_r):
                run_s, run_c, cur = c
                sl = pl.ds(i * L, L)
                qa = qa_r[sl]
                seg = seg_r[sl]
                changed = seg != cur
                plsc.addupdate_scatter(acc_s, [cur], run_s, mask=changed)
                plsc.addupdate_scatter(acc_c, [cur], run_c, mask=changed)
                run_s = jnp.where(changed, qa, run_s + qa)
                run_c = jnp.where(changed, jnp.full((L,), 1.0, jnp.float32),
                                  run_c + 1.0)
                return run_s, run_c, seg

            carry = plsc.parallel_loop(0, V, unroll=4, carry=carry)(step)
            if k + NBUF < K:
                descs[k + NBUF] = start(k + NBUF)

        run_s, run_c, cur = carry
        plsc.addupdate_scatter(acc_s, [cur], run_s)
        plsc.addupdate_scatter(acc_c, [cur], run_c)

        def start2(k):
            b = k % 2
            return (
                pltpu.async_copy(qa_hbm.at[pl.ds(base + k * C, C)],
                                 qa_bufs[b], sems[b]),
                pltpu.async_copy(seg_hbm.at[pl.ds(base + k * C, C)],
                                 seg_bufs[b], sems[b]),
            )

        in_descs = [None] * K
        in_descs[0] = start2(0)
        if K > 1:
            in_descs[1] = start2(1)
        d3 = pltpu.async_copy(q_hbm, qv, semp)

        plsc.subcore_barrier()
        pltpu.sync_copy(acc_s, sh_s.at[idx], add=True)
        pltpu.sync_copy(acc_c, sh_c.at[idx], add=True)
        plsc.subcore_barrier()

        @pl.when(sid == 0)
        def _():
            pltpu.sync_copy(sh_s, psum_hbm.at[pl.ds(cid * B, B)])
            pltpu.sync_copy(sh_c, pcnt_hbm.at[pl.ds(cid * B, B)])

        plsc.subcore_barrier()
        pltpu.semaphore_signal(xsem, 1, core_index=1 - cid)
        pl.semaphore_wait(xsem, 1)

        d1 = pltpu.async_copy(psum_hbm, ps, semp)
        d2 = pltpu.async_copy(pcnt_hbm, pc, semp)
        d1.wait()
        d2.wait()
        d3.wait()

        @plsc.parallel_loop(0, B // L, unroll=4)
        def _comb(j):
            s = jnp.zeros((L,), jnp.float32)
            n = jnp.zeros((L,), jnp.float32)
            for t in range(NC):
                s = s + ps[pl.ds(j * L + t * B, L)]
                n = n + pc[pl.ds(j * L + t * B, L)]
            sl = pl.ds(j * L, L)
            corr[sl] = (qv[sl] - s) / n
            raw[sl] = s

        out_bufs = (qa_bufs[2], qa_bufs[3])
        osems = (sems[2], sems[3])
        out_descs = [None] * K
        for k in range(K):
            b = k % 2
            in_descs[k][0].wait()
            in_descs[k][1].wait()
            if k >= 2:
                out_descs[k - 2].wait()
            qa_r = qa_bufs[b]
            seg_r = seg_bufs[b]
            ob = out_bufs[b]

            @plsc.parallel_loop(0, V, unroll=8)
            def _apply(i, qa_r=qa_r, seg_r=seg_r, ob=ob):
                sl = pl.ds(i * L, L)
                seg = seg_r[sl]
                qa = qa_r[sl]
                c = plsc.load_gather(corr, [seg])
                ob[sl] = qa + c

            out_descs[k] = pltpu.async_copy(
                ob, out_hbm.at[pl.ds(base + k * C, C)], osems[b])
            if k + 2 < K:
                in_descs[k + 2] = start2(k + 2)

        if K >= 2:
            out_descs[K - 2].wait()
        out_descs[K - 1].wait()

        @pl.when(wid == 0)
        def _():
            pltpu.sync_copy(raw, rawq_hbm)

    return fused


def kernel(Za, Qa, Q, batch_seg):
    del Za
    N = Qa.shape[0]
    B = Q.shape[0]
    assert N % NW == 0
    T = N // NW

    C = 10000
    assert T % C == 0 and C % L == 0

    qa = Qa.astype(jnp.float32)
    seg = batch_seg.astype(jnp.int32)
    q = Q.astype(jnp.float32)

    out, raw_q, _, _ = _make_fused(N, B, T, C, T // C)(qa, seg, q)
    return (out, raw_q)


# speedup vs baseline: 1.0147x; 1.0046x over previous
import functools

import jax
import jax.numpy as jnp
from jax import lax
from jax.experimental import pallas as pl
from jax.experimental.pallas import tpu as pltpu
from jax.experimental.pallas import tpu_sc as plsc

L = 16
NC = 2
NS = 16
NW = NC * NS

_CP = pltpu.CompilerParams(needs_layout_passes=False)


def _make_fused(N, B, T, C, K):
    mesh = plsc.VectorSubcoreMesh(core_axis_name="c", subcore_axis_name="s")
    V = C // L
    NBUF = 4

    @functools.partial(
        pl.kernel,
        out_type=(
            jax.ShapeDtypeStruct((N,), jnp.float32),
            jax.ShapeDtypeStruct((B,), jnp.float32),
            jax.ShapeDtypeStruct((NC * B,), jnp.float32),
            jax.ShapeDtypeStruct((NC * B,), jnp.float32),
        ),
        mesh=mesh,
        compiler_params=_CP,
        scratch_types=[
            *[pltpu.VMEM((C,), jnp.float32) for _ in range(NBUF)],
            *[pltpu.VMEM((C,), jnp.int32) for _ in range(NBUF)],
            pltpu.VMEM((B,), jnp.float32),
            pltpu.VMEM((B,), jnp.float32),
            pltpu.VMEM((B,), jnp.int32),
            pltpu.VMEM_SHARED((B,), jnp.float32),
            pltpu.VMEM_SHARED((B,), jnp.float32),
            pltpu.VMEM((NC * B,), jnp.float32),
            pltpu.VMEM((NC * B,), jnp.float32),
            pltpu.VMEM((B,), jnp.float32),
            pltpu.VMEM((B,), jnp.float32),
            pltpu.VMEM((B,), jnp.float32),
            *[pltpu.SemaphoreType.DMA for _ in range(NBUF)],
            pltpu.SemaphoreType.DMA,
            pltpu.SemaphoreType.REGULAR,
        ],
    )
    def fused(qa_hbm, seg_hbm, q_hbm, out_hbm, rawq_hbm, psum_hbm, pcnt_hbm,
              *refs):
        qa_bufs = refs[0:NBUF]
        seg_bufs = refs[NBUF:2 * NBUF]
        (acc_s, acc_c, idx, sh_s, sh_c,
         ps, pc, qv, corr, raw) = refs[2 * NBUF:2 * NBUF + 10]
        sems = refs[2 * NBUF + 10:2 * NBUF + 10 + NBUF]
        semp = refs[2 * NBUF + 10 + NBUF]
        xsem = refs[2 * NBUF + 10 + NBUF + 1]

        cid = lax.axis_index("c")
        sid = lax.axis_index("s")
        wid = cid * NS + sid
        base = wid * T

        zz = jnp.zeros((L,), jnp.float32)
        lane = lax.iota(jnp.int32, L)

        @plsc.parallel_loop(0, B // L, unroll=4)
        def _zero(j):
            acc_s[pl.ds(j * L, L)] = zz
            acc_c[pl.ds(j * L, L)] = zz
            idx[pl.ds(j * L, L)] = lane + j * L

        @pl.when(sid == 0)
        def _():
            pltpu.sync_copy(acc_s, sh_s)
            pltpu.sync_copy(acc_c, sh_c)

        def start(k):
            b = k % NBUF
            return (
                pltpu.async_copy(qa_hbm.at[pl.ds(base + k * C, C)],
                                 qa_bufs[b], sems[b]),
                pltpu.async_copy(seg_hbm.at[pl.ds(base + k * C, C)],
                                 seg_bufs[b], sems[b]),
            )

        descs = [None] * K
        for k in range(min(NBUF, K)):
            descs[k] = start(k)
        descs[0][0].wait()
        descs[0][1].wait()

        cur0 = seg_bufs[0][pl.ds(0, L)]
        carry = (jnp.zeros((L,), jnp.float32),
                 jnp.zeros((L,), jnp.float32),
                 cur0)

        for k in range(K):
            b = k % NBUF
            if k > 0:
                descs[k][0].wait()
                descs[k][1].wait()
            qa_r = qa_bufs[b]
            seg_r = seg_bufs[b]

            def step(i, c, qa_r=qa_r, seg_r=seg_r):
                run_s, run_c, cur = c
                sl = pl.ds(i * L, L)
                qa = qa_r[sl]
                seg = seg_r[sl]
                changed = seg != cur
                plsc.addupdate_scatter(acc_s, [cur], run_s, mask=changed)
                plsc.addupdate_scatter(acc_c, [cur], run_c, mask=changed)
                run_s = jnp.where(changed, qa, run_s + qa)
                run_c = jnp.where(changed, jnp.full((L,), 1.0, jnp.float32),
                                  run_c + 1.0)
                return run_s, run_c, seg

            carry = plsc.parallel_loop(0, V, unroll=2, carry=carry)(step)
            if k + NBUF < K:
                descs[k + NBUF] = start(k + NBUF)

        run_s, run_c, cur = carry
        plsc.addupdate_scatter(acc_s, [cur], run_s)
        plsc.addupdate_scatter(acc_c, [cur], run_c)

        def start2(k):
            b = k % 2
            return (
                pltpu.async_copy(qa_hbm.at[pl.ds(base + k * C, C)],
                                 qa_bufs[b], sems[b]),
                pltpu.async_copy(seg_hbm.at[pl.ds(base + k * C, C)],
                                 seg_bufs[b], sems[b]),
            )

        in_descs = [None] * K
        in_descs[0] = start2(0)
        if K > 1:
            in_descs[1] = start2(1)
        d3 = pltpu.async_copy(q_hbm, qv, semp)

        plsc.subcore_barrier()
        pltpu.sync_copy(acc_s, sh_s.at[idx], add=True)
        pltpu.sync_copy(acc_c, sh_c.at[idx], add=True)
        plsc.subcore_barrier()

        @pl.when(sid == 0)
        def _():
            pltpu.sync_copy(sh_s, psum_hbm.at[pl.ds(cid * B, B)])
            pltpu.sync_copy(sh_c, pcnt_hbm.at[pl.ds(cid * B, B)])

        plsc.subcore_barrier()
        pltpu.semaphore_signal(xsem, 1, core_index=1 - cid)
        pl.semaphore_wait(xsem, 1)

        d1 = pltpu.async_copy(psum_hbm, ps, semp)
        d2 = pltpu.async_copy(pcnt_hbm, pc, semp)
        d1.wait()
        d2.wait()
        d3.wait()

        @plsc.parallel_loop(0, B // L, unroll=4)
        def _comb(j):
            s = jnp.zeros((L,), jnp.float32)
            n = jnp.zeros((L,), jnp.float32)
            for t in range(NC):
                s = s + ps[pl.ds(j * L + t * B, L)]
                n = n + pc[pl.ds(j * L + t * B, L)]
            sl = pl.ds(j * L, L)
            corr[sl] = (qv[sl] - s) / n
            raw[sl] = s

        out_bufs = (qa_bufs[2], qa_bufs[3])
        osems = (sems[2], sems[3])
        out_descs = [None] * K
        for k in range(K):
            b = k % 2
            in_descs[k][0].wait()
            in_descs[k][1].wait()
            if k >= 2:
                out_descs[k - 2].wait()
            qa_r = qa_bufs[b]
            seg_r = seg_bufs[b]
            ob = out_bufs[b]

            @plsc.parallel_loop(0, V, unroll=4)
            def _apply(i, qa_r=qa_r, seg_r=seg_r, ob=ob):
                sl = pl.ds(i * L, L)
                seg = seg_r[sl]
                qa = qa_r[sl]
                c = plsc.load_gather(corr, [seg])
                ob[sl] = qa + c

            out_descs[k] = pltpu.async_copy(
                ob, out_hbm.at[pl.ds(base + k * C, C)], osems[b])
            if k + 2 < K:
                in_descs[k + 2] = start2(k + 2)

        if K >= 2:
            out_descs[K - 2].wait()
        out_descs[K - 1].wait()

        @pl.when(wid == 0)
        def _():
            pltpu.sync_copy(raw, rawq_hbm)

    return fused


def kernel(Za, Qa, Q, batch_seg):
    del Za
    N = Qa.shape[0]
    B = Q.shape[0]
    assert N % NW == 0
    T = N // NW

    C = 10000
    assert T % C == 0 and C % L == 0

    qa = Qa.astype(jnp.float32)
    seg = batch_seg.astype(jnp.int32)
    q = Q.astype(jnp.float32)

    out, raw_q, _, _ = _make_fused(N, B, T, C, T // C)(qa, seg, q)
    return (out, raw_q)
